# baseline (speedup 1.0000x reference)
import functools

import jax
import jax.numpy as jnp
from jax import lax
from jax.experimental import pallas as pl
from jax.experimental.pallas import tpu as pltpu
from jax.experimental.pallas import tpu_sc as plsc

_FEAT = 64
_BATCH = 16384
_NC, _NS, _L = 2, 16, 16
_NW = _NC * _NS
_BPW = _BATCH // _NW
_CHUNK = 128
_NCH = _BPW // _CHUNK


@functools.partial(
    pl.kernel,
    out_type=jax.ShapeDtypeStruct((_NW, _L), jnp.float32),
    mesh=plsc.VectorSubcoreMesh(core_axis_name="c", subcore_axis_name="s"),
    compiler_params=pltpu.CompilerParams(use_tc_tiling_on_sc=False),
    scratch_types=[
        pltpu.VMEM((_NCH, _CHUNK), jnp.int32),
        pltpu.VMEM((_BPW, _FEAT), jnp.float32),
        pltpu.VMEM((_BPW, _FEAT), jnp.float32),
        pltpu.VMEM((_L,), jnp.float32),
        pltpu.SemaphoreType.DMA,
        pltpu.SemaphoreType.DMA,
    ],
)
def _center_partials(feat_hbm, lab_hbm, cent_hbm, out_hbm,
                     idx_v, feat_v, cent_v, acc_v, fsem, gsem):
    wid = lax.axis_index("s") * _NC + lax.axis_index("c")
    base = wid * _BPW

    pltpu.sync_copy(lab_hbm.at[pl.ds(wid * _NCH, _NCH)], idx_v)
    fcp = pltpu.async_copy(feat_hbm.at[pl.ds(base, _BPW)], feat_v, fsem)
    gcps = [
        pltpu.async_copy(cent_hbm.at[idx_v.at[j]],
                         cent_v.at[pl.ds(j * _CHUNK, _CHUNK)], gsem)
        for j in range(_NCH)
    ]
    fcp.wait()
    for c in gcps:
        c.wait()

    def row(i, acc):
        for j in range(_FEAT // _L):
            f = feat_v[i, pl.ds(j * _L, _L)]
            c = cent_v[i, pl.ds(j * _L, _L)]
            d = f - c
            acc = acc + d * d
        return acc

    acc = lax.fori_loop(0, _BPW, row, jnp.zeros((_L,), jnp.float32))
    acc_v[...] = acc
    pltpu.sync_copy(acc_v, out_hbm.at[wid])


def kernel(features, labels, centers):
    lab = labels.astype(jnp.int32).reshape(_NW * _NCH, _CHUNK)
    partials = _center_partials(features, lab, centers)
    return jnp.sum(partials) / features.shape[0]


# speedup vs baseline: 1.3136x; 1.3136x over previous
import functools

import jax
import jax.numpy as jnp
from jax import lax
from jax.experimental import pallas as pl
from jax.experimental.pallas import tpu as pltpu
from jax.experimental.pallas import tpu_sc as plsc

_FEAT = 64
_BATCH = 16384
_NC, _NS, _L = 2, 16, 16
_NW = _NC * _NS
_BPW = _BATCH // _NW
_CH = 256
_NCH = _BPW // _CH


@functools.partial(
    pl.kernel,
    out_type=jax.ShapeDtypeStruct((_NW, _L), jnp.float32),
    mesh=plsc.VectorSubcoreMesh(core_axis_name="c", subcore_axis_name="s"),
    scratch_types=[
        pltpu.VMEM((_BPW,), jnp.int32),
        pltpu.VMEM((_BPW, _FEAT), jnp.float32),
        pltpu.VMEM((_CH, _FEAT), jnp.float32),
        pltpu.VMEM((_L,), jnp.float32),
        pltpu.SemaphoreType.DMA,
        pltpu.SemaphoreType.DMA,
    ],
)
def _center_partials(feat_hbm, lab_hbm, cent_hbm, out_hbm,
                     lab_v, feat_v, cent_v, acc_v, fsem, gsem):
    wid = lax.axis_index("s") * _NC + lax.axis_index("c")
    base = wid * _BPW

    pltpu.sync_copy(lab_hbm.at[pl.ds(base, _BPW)], lab_v)
    fcp = pltpu.async_copy(feat_hbm.at[pl.ds(base, _BPW)], feat_v, fsem)

    acc = jnp.zeros((_L,), jnp.float32)
    for c in range(_NCH):
        def enq(g, _, c=c):
            idx = lab_v[pl.ds(c * _CH + g * _L, _L)]
            for j in range(_L):
                lab = idx[j]
                pltpu.async_copy(cent_hbm.at[pl.ds(lab, 1)],
                                 cent_v.at[pl.ds(g * _L + j, 1)], gsem)
            return 0
        lax.fori_loop(0, _CH // _L, enq, 0)

        def drain(g, _):
            pltpu.make_async_copy(cent_hbm.at[pl.ds(0, 1)],
                                  cent_v.at[pl.ds(g, 1)], gsem).wait()
            return 0
        lax.fori_loop(0, _CH, drain, 0)
        if c == 0:
            fcp.wait()

        def row(i, a, c=c):
            for j in range(_FEAT // _L):
                f = feat_v[c * _CH + i, pl.ds(j * _L, _L)]
                g = cent_v[i, pl.ds(j * _L, _L)]
                d = f - g
                a = a + d * d
            return a
        acc = lax.fori_loop(0, _CH, row, acc)

    acc_v[...] = acc
    pltpu.sync_copy(acc_v, out_hbm.at[wid])


def kernel(features, labels, centers):
    lab = labels.astype(jnp.int32)
    partials = _center_partials(features, lab, centers)
    return jnp.sum(partials) / features.shape[0]


# speedup vs baseline: 1.8511x; 1.4091x over previous
import functools

import jax
import jax.numpy as jnp
from jax import lax
from jax.experimental import pallas as pl
from jax.experimental.pallas import tpu as pltpu
from jax.experimental.pallas import tpu_sc as plsc

_FEAT = 64
_BATCH = 16384
_CLASSES = 100000
_NC, _NS, _L = 2, 16, 16
_NW = _NC * _NS
_RPW = _FEAT // _NW
_HB = _BATCH // 2


@functools.partial(
    pl.kernel,
    out_type=jax.ShapeDtypeStruct((_NW, _L), jnp.float32),
    mesh=plsc.VectorSubcoreMesh(core_axis_name="c", subcore_axis_name="s"),
    compiler_params=pltpu.CompilerParams(needs_layout_passes=False),
    scratch_types=[
        pltpu.VMEM((_CLASSES,), jnp.float32),
        pltpu.VMEM((_BATCH,), jnp.int32),
        pltpu.VMEM((_HB,), jnp.float32),
        pltpu.VMEM((_L,), jnp.float32),
        pltpu.SemaphoreType.DMA,
        pltpu.SemaphoreType.DMA,
    ],
)
def _center_partials(feat_hbm, lab_hbm, cent_hbm, out_hbm,
                     crow, labv, frow, acc_v, csem, fsem):
    wid = lax.axis_index("s") * _NC + lax.axis_index("c")

    pltpu.sync_copy(lab_hbm, labv)

    acc = jnp.zeros((_L,), jnp.float32)
    for r in range(_RPW):
        j = wid * _RPW + r
        pltpu.async_copy(cent_hbm.at[j], crow, csem).wait()
        for h in range(2):
            pltpu.async_copy(feat_hbm.at[j, pl.ds(h * _HB, _HB)],
                             frow, fsem).wait()

            def step(g, a, h=h):
                idx = labv[pl.ds(h * _HB + g * _L, _L)]
                f = frow[pl.ds(g * _L, _L)]
                c = plsc.load_gather(crow, [idx])
                d = f - c
                return a + d * d

            acc = lax.fori_loop(0, _HB // _L, step, acc)

    acc_v[...] = acc
    pltpu.sync_copy(acc_v, out_hbm.at[wid])


def kernel(features, labels, centers):
    lab = labels.astype(jnp.int32)
    partials = _center_partials(features.T, lab, centers.T)
    return jnp.sum(partials) / features.shape[0]


# speedup vs baseline: 2.1442x; 1.1584x over previous
import functools

import jax
import jax.numpy as jnp
from jax import lax
from jax.experimental import pallas as pl
from jax.experimental.pallas import tpu as pltpu
from jax.experimental.pallas import tpu_sc as plsc

_FEAT = 64
_BATCH = 16384
_CLASSES = 100000
_NC, _NS, _L = 2, 16, 16
_NW = _NC * _NS
_RPW = _FEAT // _NW
_HB = _BATCH // 2
_UNROLL = 8


@functools.partial(
    pl.kernel,
    out_type=jax.ShapeDtypeStruct((_NW, _L), jnp.float32),
    mesh=plsc.VectorSubcoreMesh(core_axis_name="c", subcore_axis_name="s"),
    compiler_params=pltpu.CompilerParams(needs_layout_passes=False),
    scratch_types=[
        pltpu.VMEM((_CLASSES,), jnp.float32),
        pltpu.VMEM((_BATCH,), jnp.int32),
        pltpu.VMEM((_HB,), jnp.float32),
        pltpu.VMEM((_L,), jnp.float32),
        pltpu.SemaphoreType.DMA,
        pltpu.SemaphoreType.DMA,
    ],
)
def _center_partials(feat_hbm, lab_hbm, cent_hbm, out_hbm,
                     crow, labv, frow, acc_v, csem, fsem):
    wid = lax.axis_index("s") * _NC + lax.axis_index("c")

    pltpu.sync_copy(lab_hbm, labv)

    acc = jnp.zeros((_L,), jnp.float32)
    for r in range(_RPW):
        j = wid * _RPW + r
        pltpu.async_copy(cent_hbm.at[j], crow, csem).wait()
        for h in range(2):
            pltpu.async_copy(feat_hbm.at[j, pl.ds(h * _HB, _HB)],
                             frow, fsem).wait()

            def step(g, a, h=h):
                for u in range(_UNROLL):
                    o = g * _L * _UNROLL + u * _L
                    idx = labv[pl.ds(h * _HB + o, _L)]
                    f = frow[pl.ds(o, _L)]
                    c = plsc.load_gather(crow, [idx])
                    d = f - c
                    a = a + d * d
                return a

            acc = lax.fori_loop(0, _HB // (_L * _UNROLL), step, acc)

    acc_v[...] = acc
    pltpu.sync_copy(acc_v, out_hbm.at[wid])


def kernel(features, labels, centers):
    lab = labels.astype(jnp.int32)
    partials = _center_partials(features.T, lab, centers.T)
    return jnp.sum(partials) / features.shape[0]


# speedup vs baseline: 2.1594x; 1.0071x over previous
import functools

import jax
import jax.numpy as jnp
from jax import lax
from jax.experimental import pallas as pl
from jax.experimental.pallas import tpu as pltpu
from jax.experimental.pallas import tpu_sc as plsc

_FEAT = 64
_BATCH = 16384
_CLASSES = 100000
_NC, _NS, _L = 2, 16, 16
_NW = _NC * _NS
_RPW = _FEAT // _NW
_HB = _BATCH // 2
_UNROLL = 8


@functools.partial(
    pl.kernel,
    out_type=jax.ShapeDtypeStruct((_NW, _L), jnp.float32),
    mesh=plsc.VectorSubcoreMesh(core_axis_name="c", subcore_axis_name="s"),
    compiler_params=pltpu.CompilerParams(needs_layout_passes=False),
    scratch_types=[
        pltpu.VMEM((_CLASSES,), jnp.float32),
        pltpu.VMEM((_BATCH,), jnp.int32),
        pltpu.VMEM((_HB,), jnp.float32),
        pltpu.VMEM((_L,), jnp.float32),
        pltpu.SemaphoreType.DMA,
        pltpu.SemaphoreType.DMA,
    ],
)
def _center_partials(feat_hbm, lab_hbm, cent_hbm, out_hbm,
                     crow, labv, frow, acc_v, csem, fsem):
    wid = lax.axis_index("s") * _NC + lax.axis_index("c")

    pltpu.sync_copy(lab_hbm, labv)

    accs = tuple(jnp.zeros((_L,), jnp.float32) for _ in range(_UNROLL))
    for r in range(_RPW):
        j = wid * _RPW + r
        pltpu.async_copy(cent_hbm.at[j], crow, csem).wait()
        for h in range(2):
            pltpu.async_copy(feat_hbm.at[j, pl.ds(h * _HB, _HB)],
                             frow, fsem).wait()

            def step(g, a, h=h):
                out = []
                for u in range(_UNROLL):
                    o = g * _L * _UNROLL + u * _L
                    idx = labv[pl.ds(h * _HB + o, _L)]
                    f = frow[pl.ds(o, _L)]
                    c = plsc.load_gather(crow, [idx])
                    d = f - c
                    out.append(a[u] + d * d)
                return tuple(out)

            accs = lax.fori_loop(0, _HB // (_L * _UNROLL), step, accs)

    acc_v[...] = functools.reduce(lambda x, y: x + y, accs)
    pltpu.sync_copy(acc_v, out_hbm.at[wid])


def kernel(features, labels, centers):
    lab = labels.astype(jnp.int32)
    partials = _center_partials(features.T, lab, centers.T)
    return jnp.sum(partials) / features.shape[0]
